# baseline (speedup 1.0000x reference)
import functools

import jax
import jax.numpy as jnp
from jax.experimental import pallas as pl
from jax.experimental.pallas import tpu as pltpu

_ZERO_WEIGHTING = 0.5
_NONZERO_WEIGHTING = 1.0

_ROWS = 18816
_COLS = 1024
_BLOCK_ROWS = 1176


def _loss_body(rec_ref, tgt_ref, out_ref, acc_ref, *, total_n):
    i = pl.program_id(0)
    n = pl.num_programs(0)

    t = tgt_ref[...]
    r = rec_ref[...]
    zero = t == 0.0
    d = r - t
    ssq = jnp.sum(jnp.where(zero, d * d, 0.0))
    sab = jnp.sum(jnp.where(zero, 0.0, jnp.abs(d)))
    nz = jnp.sum(zero.astype(jnp.float32))

    @pl.when(i == 0)
    def _init():
        acc_ref[0] = 0.0
        acc_ref[1] = 0.0
        acc_ref[2] = 0.0

    acc_ref[0] += ssq
    acc_ref[1] += sab
    acc_ref[2] += nz

    @pl.when(i == n - 1)
    def _finish():
        n_zero = acc_ref[2]
        n_nonzero = total_n - n_zero
        zero_loss = jnp.where(n_zero > 0, acc_ref[0] / jnp.maximum(n_zero, 1.0), 0.0)
        nonzero_loss = jnp.where(
            n_nonzero > 0, acc_ref[1] / jnp.maximum(n_nonzero, 1.0), 0.0
        )
        out_ref[0] = _ZERO_WEIGHTING * zero_loss + _NONZERO_WEIGHTING * nonzero_loss


def kernel(reconstructed_image, target_image):
    total_n = float(reconstructed_image.size)
    rec = reconstructed_image.reshape(_ROWS, _COLS)
    tgt = target_image.reshape(_ROWS, _COLS)

    grid = _ROWS // _BLOCK_ROWS
    out = pl.pallas_call(
        functools.partial(_loss_body, total_n=total_n),
        grid=(grid,),
        in_specs=[
            pl.BlockSpec((_BLOCK_ROWS, _COLS), lambda i: (i, 0)),
            pl.BlockSpec((_BLOCK_ROWS, _COLS), lambda i: (i, 0)),
        ],
        out_specs=pl.BlockSpec(memory_space=pltpu.SMEM),
        out_shape=jax.ShapeDtypeStruct((1,), jnp.float32),
        scratch_shapes=[pltpu.SMEM((3,), jnp.float32)],
    )(rec, tgt)
    return out[0]


# speedup vs baseline: 4.2205x; 4.2205x over previous
import functools

import jax
import jax.numpy as jnp
from jax.experimental import pallas as pl
from jax.experimental.pallas import tpu as pltpu

_ZERO_WEIGHTING = 0.5
_NONZERO_WEIGHTING = 1.0

_PLANES = 384
_H = 224
_W = 224
_BLOCK_PLANES = 24


def _loss_body(rec_ref, tgt_ref, out_ref, acc_ref, *, total_n):
    i = pl.program_id(0)
    n = pl.num_programs(0)

    t = tgt_ref[...]
    r = rec_ref[...]
    zero = t == 0.0
    d = r - t
    ssq = jnp.sum(jnp.where(zero, d * d, 0.0))
    sab = jnp.sum(jnp.where(zero, 0.0, jnp.abs(d)))
    nz = jnp.sum(zero.astype(jnp.float32))

    @pl.when(i == 0)
    def _init():
        acc_ref[0] = 0.0
        acc_ref[1] = 0.0
        acc_ref[2] = 0.0

    acc_ref[0] += ssq
    acc_ref[1] += sab
    acc_ref[2] += nz

    @pl.when(i == n - 1)
    def _finish():
        n_zero = acc_ref[2]
        n_nonzero = total_n - n_zero
        zero_loss = jnp.where(n_zero > 0, acc_ref[0] / jnp.maximum(n_zero, 1.0), 0.0)
        nonzero_loss = jnp.where(
            n_nonzero > 0, acc_ref[1] / jnp.maximum(n_nonzero, 1.0), 0.0
        )
        out_ref[0] = _ZERO_WEIGHTING * zero_loss + _NONZERO_WEIGHTING * nonzero_loss


def kernel(reconstructed_image, target_image):
    total_n = float(reconstructed_image.size)
    rec = reconstructed_image.reshape(_PLANES, _H, _W)
    tgt = target_image.reshape(_PLANES, _H, _W)

    grid = _PLANES // _BLOCK_PLANES
    out = pl.pallas_call(
        functools.partial(_loss_body, total_n=total_n),
        grid=(grid,),
        in_specs=[
            pl.BlockSpec((_BLOCK_PLANES, _H, _W), lambda i: (i, 0, 0)),
            pl.BlockSpec((_BLOCK_PLANES, _H, _W), lambda i: (i, 0, 0)),
        ],
        out_specs=pl.BlockSpec(memory_space=pltpu.SMEM),
        out_shape=jax.ShapeDtypeStruct((1,), jnp.float32),
        scratch_shapes=[pltpu.SMEM((3,), jnp.float32)],
    )(rec, tgt)
    return out[0]
